# baseline (speedup 1.0000x reference)
import jax
import jax.numpy as jnp
from jax.experimental import pallas as pl
from jax.experimental.pallas import tpu as pltpu


def _body(xq_ref, xsel_ref, xpair_ref, sxsel_ref, sxpair_ref, sfpair_ref,
          sx_ref, out_ref, snorm_ref, wmat_ref):
    S = sx_ref.shape[0]
    TN = xq_ref.shape[1]
    f32 = jnp.float32
    i = pl.program_id(0)
    nt = pl.num_programs(0) - 1

    @pl.when(i > 0)
    def _():
        out_ref[:, :] = jax.lax.dot_general(
            sfpair_ref[:, :], wmat_ref[:, :], (((1,), (0,)), ((), ())),
            preferred_element_type=f32,
        )

    @pl.when(i == 0)
    def _():
        sn = jnp.zeros((S, 1), f32)
        for c in range(3):
            sc = sx_ref[:, c : c + 1]
            sn = sn + sc * sc
        snorm_ref[:, :] = sn

    @pl.when(i < nt)
    def _():
        snorm = snorm_ref[:, :]
        xnorm = jnp.zeros((1, TN), f32)
        for c in range(3):
            xc = xq_ref[c : c + 1, :]
            xnorm = xnorm + xc * xc

        dotb = jax.lax.dot_general(
            sxsel_ref[:, :], xsel_ref[:, :], (((1,), (0,)), ((), ())),
            preferred_element_type=f32,
        )
        d_sel = (-2.0 * dotb + xnorm) + snorm

        dcorr = jax.lax.dot_general(
            sxpair_ref[:, :], xpair_ref[:, :], (((1,), (0,)), ((), ())),
            preferred_element_type=f32,
        )
        d_exact = d_sel - (dcorr + dcorr)

        big = jnp.float32(jnp.inf)
        exact_vals = []
        for k in range(3):
            v = jnp.min(d_sel, axis=0, keepdims=True)
            eq = d_sel == v
            exact_vals.append(
                jnp.min(jnp.where(eq, d_exact, big), axis=0, keepdims=True)
            )
            if k < 2:
                d_sel = jnp.where(eq, big, d_sel)

        invs = [
            1.0 / jnp.maximum(jnp.sqrt(jnp.maximum(v, 0.0)), 1e-10)
            for v in exact_vals
        ]
        norm = invs[0] + invs[1] + invs[2]

        bf16 = jnp.bfloat16
        w = [iv / norm for iv in invs]
        wh = [x.astype(bf16) for x in w]
        wl = [(x - y.astype(f32)).astype(bf16) for x, y in zip(w, wh)]
        wmat_ref[:, :] = jnp.concatenate(wh + wh + wl, axis=0)


def _split_bf16(a):
    hi_f32 = jax.lax.reduce_precision(a, exponent_bits=8, mantissa_bits=7)
    hi = hi_f32.astype(jnp.bfloat16)
    lo = (a - hi_f32).astype(jnp.bfloat16)
    return hi, lo


@jax.jit
def kernel(xyz, sparse_xyz, sparse_frame):
    B, C, N = xyz.shape
    S = sparse_xyz.shape[2]
    TN = 512
    f32 = jnp.float32

    xq = xyz[0]
    sx = jnp.transpose(sparse_xyz[0])
    sf = jnp.transpose(sparse_frame[0])

    xh, xl = _split_bf16(xq)
    sxh, sxl = _split_bf16(sx)
    sfh, sfl = _split_bf16(sf)
    xsel = xh
    sxsel = sxh
    xpair = jnp.concatenate([xh, xl], axis=0)
    sxpair = jnp.concatenate([sxl, sxh], axis=1)
    sfpair = jnp.concatenate([sfh, sfl, sfh], axis=1)

    NT = N // TN

    def xmap(i):
        return (0, jnp.where(i < NT, i, NT - 1))

    def omap(i):
        return (0, jnp.where(i > 0, i - 1, 0))

    out = pl.pallas_call(
        _body,
        grid=(NT + 1,),
        in_specs=[
            pl.BlockSpec((3, TN), xmap),
            pl.BlockSpec((3, TN), xmap),
            pl.BlockSpec((6, TN), xmap),
            pl.BlockSpec((S, 3), lambda i: (0, 0)),
            pl.BlockSpec((S, 6), lambda i: (0, 0)),
            pl.BlockSpec((S, 9), lambda i: (0, 0)),
            pl.BlockSpec((S, 3), lambda i: (0, 0)),
        ],
        out_specs=pl.BlockSpec((S, TN), omap),
        out_shape=jax.ShapeDtypeStruct((S, N), f32),
        scratch_shapes=[
            pltpu.VMEM((S, 1), f32),
            pltpu.VMEM((9, TN), jnp.bfloat16),
        ],
    )(xq, xsel, xpair, sxsel, sxpair, sfpair, sx)
    return out[None]
